# Initial kernel scaffold; baseline (speedup 1.0000x reference)
import jax
import jax.numpy as jnp
from jax.experimental import pallas as pl


def kernel(hidden_states, W):
    raise NotImplementedError("write your pallas kernel here")


# speedup vs baseline: 1.3857x; 1.3857x over previous
import functools

import jax
import jax.numpy as jnp
from jax.experimental import pallas as pl
from jax.experimental.pallas import tpu as pltpu

HIDDEN = 4096
NUM_EXPERTS = 64
TOP_K = 8
TOKEN_BLOCK = 512


def _router_kernel(x_ref, w_ref, topw_ref, topi_ref, lbl_ref, zl_ref,
                   util_ref, probsum_ref, *, num_tokens, num_steps):
    i = pl.program_id(0)

    @pl.when(i == 0)
    def _init():
        zl_ref[...] = jnp.zeros_like(zl_ref)
        util_ref[...] = jnp.zeros_like(util_ref)
        probsum_ref[...] = jnp.zeros_like(probsum_ref)
        lbl_ref[...] = jnp.zeros_like(lbl_ref)

    x = x_ref[...]
    w = w_ref[...]
    logits = jax.lax.dot_general(
        x, w, (((1,), (1,)), ((), ())), preferred_element_type=jnp.float32)

    m = jnp.max(logits, axis=1, keepdims=True)
    e = jnp.exp(logits - m)
    s = jnp.sum(e, axis=1, keepdims=True)
    probs = e / s

    lse = m + jnp.log(s)
    zl_ref[...] += jnp.sum(lse * lse).reshape(1, 1)

    probsum_ref[...] += jnp.sum(probs, axis=0, keepdims=True)

    iota = jax.lax.broadcasted_iota(jnp.int32, probs.shape, 1)
    p = probs
    top_w = []
    top_i = []
    for _ in range(TOP_K):
        cur = jnp.max(p, axis=1, keepdims=True)
        idx = jnp.min(jnp.where(p == cur, iota, NUM_EXPERTS), axis=1,
                      keepdims=True)
        top_w.append(cur)
        top_i.append(idx)
        p = jnp.where(iota == idx, -1.0, p)

    topw = jnp.concatenate(top_w, axis=1)
    topi = jnp.concatenate(top_i, axis=1)
    topw_ref[...] = topw / jnp.sum(topw, axis=1, keepdims=True)
    topi_ref[...] = topi

    top1 = top_i[0]
    util_ref[...] += jnp.sum((iota == top1).astype(jnp.float32), axis=0,
                             keepdims=True)

    @pl.when(i == num_steps - 1)
    def _finalize():
        counts = util_ref[...]
        probsum = probsum_ref[...]
        inv_n = 1.0 / num_tokens
        lbl_ref[...] = ((NUM_EXPERTS * inv_n * inv_n)
                        * jnp.sum(counts * probsum)).reshape(1, 1)
        zl_ref[...] = zl_ref[...] * inv_n
        util_ref[...] = counts * inv_n


def kernel(hidden_states, W):
    B, S, H = hidden_states.shape
    x = hidden_states.reshape(-1, H)
    num_tokens = x.shape[0]
    num_steps = num_tokens // TOKEN_BLOCK

    grid = (num_steps,)
    kern = functools.partial(_router_kernel, num_tokens=num_tokens,
                             num_steps=num_steps)
    topw, topi, lbl, zl, util = pl.pallas_call(
        kern,
        grid=grid,
        in_specs=[
            pl.BlockSpec((TOKEN_BLOCK, H), lambda i: (i, 0)),
            pl.BlockSpec((NUM_EXPERTS, H), lambda i: (0, 0)),
        ],
        out_specs=[
            pl.BlockSpec((TOKEN_BLOCK, TOP_K), lambda i: (i, 0)),
            pl.BlockSpec((TOKEN_BLOCK, TOP_K), lambda i: (i, 0)),
            pl.BlockSpec((1, 1), lambda i: (0, 0)),
            pl.BlockSpec((1, 1), lambda i: (0, 0)),
            pl.BlockSpec((1, NUM_EXPERTS), lambda i: (0, 0)),
        ],
        out_shape=[
            jax.ShapeDtypeStruct((num_tokens, TOP_K), jnp.float32),
            jax.ShapeDtypeStruct((num_tokens, TOP_K), jnp.int32),
            jax.ShapeDtypeStruct((1, 1), jnp.float32),
            jax.ShapeDtypeStruct((1, 1), jnp.float32),
            jax.ShapeDtypeStruct((1, NUM_EXPERTS), jnp.float32),
        ],
        scratch_shapes=[pltpu.VMEM((1, NUM_EXPERTS), jnp.float32)],
        compiler_params=pltpu.CompilerParams(
            dimension_semantics=("arbitrary",)),
    )(x, W)

    return (topw, topi, lbl.reshape(()), zl.reshape(()), util.reshape(-1))


# speedup vs baseline: 1.5265x; 1.1016x over previous
import functools

import jax
import jax.numpy as jnp
from jax.experimental import pallas as pl
from jax.experimental.pallas import tpu as pltpu

HIDDEN = 4096
NUM_EXPERTS = 64
TOP_K = 8
TOKEN_BLOCK = 512


def _router_kernel(x_ref, w_ref, topw_ref, topi_ref, lbl_ref, zl_ref,
                   util_ref, probsum_ref, *, num_tokens, num_steps):
    i = pl.program_id(0)

    @pl.when(i == 0)
    def _init():
        zl_ref[...] = jnp.zeros_like(zl_ref)
        util_ref[...] = jnp.zeros_like(util_ref)
        probsum_ref[...] = jnp.zeros_like(probsum_ref)
        lbl_ref[...] = jnp.zeros_like(lbl_ref)

    x = x_ref[...]
    w = w_ref[...]
    logits = jax.lax.dot_general(
        x, w, (((1,), (1,)), ((), ())), preferred_element_type=jnp.float32)

    m = jnp.max(logits, axis=1, keepdims=True)
    e = jnp.exp(logits - m)
    s = jnp.sum(e, axis=1, keepdims=True)
    probs = e / s

    lse = m + jnp.log(s)
    zl_ref[...] += jnp.sum(lse * lse).reshape(1, 1)

    probsum_ref[...] += jnp.sum(probs, axis=0, keepdims=True)

    iota = jax.lax.broadcasted_iota(jnp.int32, probs.shape, 1)
    bits = jax.lax.bitcast_convert_type(probs, jnp.int32)
    key = (bits & ~63) | (NUM_EXPERTS - 1 - iota)
    top_keys = []
    for _ in range(TOP_K):
        cur = jnp.max(key, axis=1, keepdims=True)
        top_keys.append(cur)
        key = jnp.where(key == cur, jnp.int32(-2**31), key)

    topk = jnp.concatenate(top_keys, axis=1)
    topi = (NUM_EXPERTS - 1) - (topk & 63)
    topw = jax.lax.bitcast_convert_type(topk & ~63, jnp.float32)
    topw_ref[...] = topw / jnp.sum(topw, axis=1, keepdims=True)
    topi_ref[...] = topi

    top1 = top_keys[0]
    top1_idx = (NUM_EXPERTS - 1) - (top1 & 63)
    util_ref[...] += jnp.sum((iota == top1_idx).astype(jnp.float32), axis=0,
                             keepdims=True)

    @pl.when(i == num_steps - 1)
    def _finalize():
        counts = util_ref[...]
        probsum = probsum_ref[...]
        inv_n = 1.0 / num_tokens
        lbl_ref[...] = ((NUM_EXPERTS * inv_n * inv_n)
                        * jnp.sum(counts * probsum)).reshape(1, 1)
        zl_ref[...] = zl_ref[...] * inv_n
        util_ref[...] = counts * inv_n


def kernel(hidden_states, W):
    B, S, H = hidden_states.shape
    x = hidden_states.reshape(-1, H)
    num_tokens = x.shape[0]
    num_steps = num_tokens // TOKEN_BLOCK

    grid = (num_steps,)
    kern = functools.partial(_router_kernel, num_tokens=num_tokens,
                             num_steps=num_steps)
    topw, topi, lbl, zl, util = pl.pallas_call(
        kern,
        grid=grid,
        in_specs=[
            pl.BlockSpec((TOKEN_BLOCK, H), lambda i: (i, 0)),
            pl.BlockSpec((NUM_EXPERTS, H), lambda i: (0, 0)),
        ],
        out_specs=[
            pl.BlockSpec((TOKEN_BLOCK, TOP_K), lambda i: (i, 0)),
            pl.BlockSpec((TOKEN_BLOCK, TOP_K), lambda i: (i, 0)),
            pl.BlockSpec((1, 1), lambda i: (0, 0)),
            pl.BlockSpec((1, 1), lambda i: (0, 0)),
            pl.BlockSpec((1, NUM_EXPERTS), lambda i: (0, 0)),
        ],
        out_shape=[
            jax.ShapeDtypeStruct((num_tokens, TOP_K), jnp.float32),
            jax.ShapeDtypeStruct((num_tokens, TOP_K), jnp.int32),
            jax.ShapeDtypeStruct((1, 1), jnp.float32),
            jax.ShapeDtypeStruct((1, 1), jnp.float32),
            jax.ShapeDtypeStruct((1, NUM_EXPERTS), jnp.float32),
        ],
        scratch_shapes=[pltpu.VMEM((1, NUM_EXPERTS), jnp.float32)],
        compiler_params=pltpu.CompilerParams(
            dimension_semantics=("arbitrary",)),
    )(x, W)

    return (topw, topi, lbl.reshape(()), zl.reshape(()), util.reshape(-1))


# speedup vs baseline: 1.7019x; 1.1149x over previous
import functools

import jax
import jax.numpy as jnp
from jax.experimental import pallas as pl
from jax.experimental.pallas import tpu as pltpu

HIDDEN = 4096
NUM_EXPERTS = 64
TOP_K = 8
TOKEN_BLOCK = 1024


def _router_kernel(x_ref, w_ref, topw_ref, topi_ref, lbl_ref, zl_ref,
                   util_ref, probsum_ref, *, num_tokens, num_steps):
    i = pl.program_id(0)

    @pl.when(i == 0)
    def _init():
        zl_ref[...] = jnp.zeros_like(zl_ref)
        util_ref[...] = jnp.zeros_like(util_ref)
        probsum_ref[...] = jnp.zeros_like(probsum_ref)
        lbl_ref[...] = jnp.zeros_like(lbl_ref)

    x = x_ref[...]
    w = w_ref[...]
    logits = jax.lax.dot_general(
        x, w, (((1,), (1,)), ((), ())), preferred_element_type=jnp.float32)

    m = jnp.max(logits, axis=1, keepdims=True)
    e = jnp.exp(logits - m)
    s = jnp.sum(e, axis=1, keepdims=True)
    probs = e / s

    lse = m + jnp.log(s)
    zl_ref[...] += jnp.sum(lse * lse).reshape(1, 1)

    probsum_ref[...] += jnp.sum(probs, axis=0, keepdims=True)

    iota = jax.lax.broadcasted_iota(jnp.int32, probs.shape, 1)
    bits = jax.lax.bitcast_convert_type(probs, jnp.int32)
    key = (bits & ~63) | (NUM_EXPERTS - 1 - iota)
    top_keys = []
    for _ in range(TOP_K):
        cur = jnp.max(key, axis=1, keepdims=True)
        top_keys.append(cur)
        key = jnp.where(key == cur, jnp.int32(-2**31), key)

    topk = jnp.concatenate(top_keys, axis=1)
    topi = (NUM_EXPERTS - 1) - (topk & 63)
    topw = jax.lax.bitcast_convert_type(topk & ~63, jnp.float32)
    topw_ref[...] = topw / jnp.sum(topw, axis=1, keepdims=True)
    topi_ref[...] = topi

    top1 = top_keys[0]
    top1_idx = (NUM_EXPERTS - 1) - (top1 & 63)
    util_ref[...] += jnp.sum((iota == top1_idx).astype(jnp.float32), axis=0,
                             keepdims=True)

    @pl.when(i == num_steps - 1)
    def _finalize():
        counts = util_ref[...]
        probsum = probsum_ref[...]
        inv_n = 1.0 / num_tokens
        lbl_ref[...] = ((NUM_EXPERTS * inv_n * inv_n)
                        * jnp.sum(counts * probsum)).reshape(1, 1)
        zl_ref[...] = zl_ref[...] * inv_n
        util_ref[...] = counts * inv_n


def kernel(hidden_states, W):
    B, S, H = hidden_states.shape
    x = hidden_states.reshape(-1, H)
    num_tokens = x.shape[0]
    num_steps = num_tokens // TOKEN_BLOCK

    grid = (num_steps,)
    kern = functools.partial(_router_kernel, num_tokens=num_tokens,
                             num_steps=num_steps)
    topw, topi, lbl, zl, util = pl.pallas_call(
        kern,
        grid=grid,
        in_specs=[
            pl.BlockSpec((TOKEN_BLOCK, H), lambda i: (i, 0)),
            pl.BlockSpec((NUM_EXPERTS, H), lambda i: (0, 0)),
        ],
        out_specs=[
            pl.BlockSpec((TOKEN_BLOCK, TOP_K), lambda i: (i, 0)),
            pl.BlockSpec((TOKEN_BLOCK, TOP_K), lambda i: (i, 0)),
            pl.BlockSpec((1, 1), lambda i: (0, 0)),
            pl.BlockSpec((1, 1), lambda i: (0, 0)),
            pl.BlockSpec((1, NUM_EXPERTS), lambda i: (0, 0)),
        ],
        out_shape=[
            jax.ShapeDtypeStruct((num_tokens, TOP_K), jnp.float32),
            jax.ShapeDtypeStruct((num_tokens, TOP_K), jnp.int32),
            jax.ShapeDtypeStruct((1, 1), jnp.float32),
            jax.ShapeDtypeStruct((1, 1), jnp.float32),
            jax.ShapeDtypeStruct((1, NUM_EXPERTS), jnp.float32),
        ],
        scratch_shapes=[pltpu.VMEM((1, NUM_EXPERTS), jnp.float32)],
        compiler_params=pltpu.CompilerParams(
            dimension_semantics=("arbitrary",)),
    )(x, W)

    return (topw, topi, lbl.reshape(()), zl.reshape(()), util.reshape(-1))


# speedup vs baseline: 1.8450x; 1.0841x over previous
import functools

import jax
import jax.numpy as jnp
from jax.experimental import pallas as pl
from jax.experimental.pallas import tpu as pltpu

HIDDEN = 4096
NUM_EXPERTS = 64
TOP_K = 8
TOKEN_BLOCK = 1024
CHUNK = 256


def _router_kernel(x_ref, w_ref, topw_ref, topi_ref, lbl_ref, zl_ref,
                   util_ref, probsum_ref, *, num_tokens, num_steps):
    i = pl.program_id(0)

    @pl.when(i == 0)
    def _init():
        zl_ref[...] = jnp.zeros_like(zl_ref)
        util_ref[...] = jnp.zeros_like(util_ref)
        probsum_ref[...] = jnp.zeros_like(probsum_ref)
        lbl_ref[...] = jnp.zeros_like(lbl_ref)

    w = w_ref[...]
    acc_z = jnp.zeros((1, 1), jnp.float32)
    acc_probsum = jnp.zeros((NUM_EXPERTS, 1), jnp.float32)
    acc_counts = jnp.zeros((NUM_EXPERTS, 1), jnp.float32)

    for c in range(TOKEN_BLOCK // CHUNK):
        sl = pl.ds(c * CHUNK, CHUNK)
        x = x_ref[sl, :]
        lt = jax.lax.dot_general(
            w, x, (((1,), (1,)), ((), ())), preferred_element_type=jnp.float32)

        m = jnp.max(lt, axis=0, keepdims=True)
        e = jnp.exp(lt - m)
        s = jnp.sum(e, axis=0, keepdims=True)
        probs = e / s

        lse = m + jnp.log(s)
        acc_z += jnp.sum(lse * lse).reshape(1, 1)

        acc_probsum += jnp.sum(probs, axis=1, keepdims=True)

        iota = jax.lax.broadcasted_iota(jnp.int32, probs.shape, 0)
        bits = jax.lax.bitcast_convert_type(probs, jnp.int32)
        key = (bits & ~63) | (NUM_EXPERTS - 1 - iota)
        top_keys = []
        for _ in range(TOP_K):
            cur = jnp.max(key, axis=0, keepdims=True)
            top_keys.append(cur)
            key = jnp.where(key == cur, jnp.int32(-2**31), key)

        topk = jnp.concatenate(top_keys, axis=0)
        topi = (NUM_EXPERTS - 1) - (topk & 63)
        topw = jax.lax.bitcast_convert_type(topk & ~63, jnp.float32)
        topw = topw / jnp.sum(topw, axis=0, keepdims=True)
        topw_ref[sl, :] = topw.T
        topi_ref[sl, :] = topi.T

        top1_idx = topi[0:1, :]
        acc_counts += jnp.sum((iota == top1_idx).astype(jnp.float32), axis=1,
                              keepdims=True)

    zl_ref[...] += acc_z
    probsum_ref[...] += acc_probsum
    util_ref[...] += acc_counts

    @pl.when(i == num_steps - 1)
    def _finalize():
        counts = util_ref[...]
        probsum = probsum_ref[...]
        inv_n = 1.0 / num_tokens
        lbl_ref[...] = ((NUM_EXPERTS * inv_n * inv_n)
                        * jnp.sum(counts * probsum)).reshape(1, 1)
        zl_ref[...] = zl_ref[...] * inv_n
        util_ref[...] = counts * inv_n


def kernel(hidden_states, W):
    B, S, H = hidden_states.shape
    x = hidden_states.reshape(-1, H)
    num_tokens = x.shape[0]
    num_steps = num_tokens // TOKEN_BLOCK

    grid = (num_steps,)
    kern = functools.partial(_router_kernel, num_tokens=num_tokens,
                             num_steps=num_steps)
    topw, topi, lbl, zl, util = pl.pallas_call(
        kern,
        grid=grid,
        in_specs=[
            pl.BlockSpec((TOKEN_BLOCK, H), lambda i: (i, 0)),
            pl.BlockSpec((NUM_EXPERTS, H), lambda i: (0, 0)),
        ],
        out_specs=[
            pl.BlockSpec((TOKEN_BLOCK, TOP_K), lambda i: (i, 0)),
            pl.BlockSpec((TOKEN_BLOCK, TOP_K), lambda i: (i, 0)),
            pl.BlockSpec((1, 1), lambda i: (0, 0)),
            pl.BlockSpec((1, 1), lambda i: (0, 0)),
            pl.BlockSpec((NUM_EXPERTS, 1), lambda i: (0, 0)),
        ],
        out_shape=[
            jax.ShapeDtypeStruct((num_tokens, TOP_K), jnp.float32),
            jax.ShapeDtypeStruct((num_tokens, TOP_K), jnp.int32),
            jax.ShapeDtypeStruct((1, 1), jnp.float32),
            jax.ShapeDtypeStruct((1, 1), jnp.float32),
            jax.ShapeDtypeStruct((NUM_EXPERTS, 1), jnp.float32),
        ],
        scratch_shapes=[pltpu.VMEM((NUM_EXPERTS, 1), jnp.float32)],
        compiler_params=pltpu.CompilerParams(
            dimension_semantics=("arbitrary",)),
    )(x, W)

    return (topw, topi, lbl.reshape(()), zl.reshape(()), util.reshape(-1))
